# baseline (speedup 1.0000x reference)
import functools

import jax
import jax.numpy as jnp
from jax import lax
from jax.experimental import pallas as pl
from jax.experimental.pallas import tpu as pltpu
from jax.experimental.pallas import tpu_sc as plsc

B, S, D = 1024, 200, 64

_info = plsc.get_sparse_core_info()
NC, NS = _info.num_cores, _info.num_subcores
NW = NC * NS
BW = B // NW
SC_ = 8
NCHUNK = S // SC_
CT = SC_ * BW

_mesh = plsc.VectorSubcoreMesh(core_axis_name="c", subcore_axis_name="s")


@functools.partial(
    pl.kernel,
    out_type=jax.ShapeDtypeStruct((S, 8, 8, 8, 128), jnp.float32),
    mesh=_mesh,
    compiler_params=pltpu.CompilerParams(
        use_tc_tiling_on_sc=False, needs_layout_passes=False),
    scratch_types=[
        pltpu.VMEM((BW * S,), jnp.int32),
        pltpu.VMEM((S, D), jnp.float32),
        pltpu.VMEM((CT, D), jnp.float32),
        pltpu.VMEM((CT, D), jnp.float32),
        pltpu.VMEM((SC_, 8, 8, BW), jnp.float32),
        pltpu.VMEM((SC_, 8, 8, BW), jnp.float32),
        pltpu.VMEM((CT,), jnp.int32),
        pltpu.VMEM((CT,), jnp.int32),
        pltpu.SemaphoreType.DMA,
        pltpu.SemaphoreType.DMA,
        pltpu.SemaphoreType.DMA,
        pltpu.SemaphoreType.DMA,
    ],
)
def _embed(x_hbm, tok_hbm, pos_hbm, out_hbm, idx_v, pos_v, gb0, gb1,
           vb0, vb1, cl0, cl1, gsem0, gsem1, ssem0, ssem1):
    wid = lax.axis_index("s") * NC + lax.axis_index("c")
    bt0 = wid // 4
    bi0 = (wid % 4) * BW

    pltpu.sync_copy(pos_hbm, pos_v)
    pltpu.sync_copy(x_hbm.at[pl.ds(wid * BW * S, BW * S)], idx_v)

    gbs = (gb0, gb1)
    vbs = (vb0, vb1)
    cls = (cl0, cl1)
    gsems = (gsem0, gsem1)
    ssems = (ssem0, ssem1)

    lanes = lax.iota(jnp.int32, 16)
    apat_idx = lanes * S
    dt_vecs = [lax.shift_right_logical(lanes, 3) + 2 * k for k in range(4)]
    di_vec = lax.bitwise_and(lanes, 7)
    zero_vec = lanes * 0

    def prep_clist(c):
        cl = cls[c % 2]
        s0 = c * SC_

        def body(g, carry):
            addr = apat_idx + ((g % 2) * 16 * S + s0 + g // 2)
            cl[pl.ds(g * 16, 16)] = plsc.load_gather(idx_v, [addr])
            return carry
        lax.fori_loop(0, CT // 16, body, 0)

    def start_gather(c):
        b = c % 2
        d0 = pltpu.async_copy(
            tok_hbm.at[cls[b].at[pl.ds(0, 128)]],
            gbs[b].at[pl.ds(0, 128)], gsems[b])
        d1 = pltpu.async_copy(
            tok_hbm.at[cls[b].at[pl.ds(128, 128)]],
            gbs[b].at[pl.ds(128, 128)], gsems[b])
        return (d0, d1)

    def compute(c):
        b = c % 2
        gb, vb = gbs[b], vbs[b]
        s0 = c * SC_

        def addbody(i, carry):
            sl = lax.shift_right_logical(i, 5)
            for k in range(4):
                sli = pl.ds(k * 16, 16)
                plsc.addupdate(gb.at[i, sli], pos_v[s0 + sl, sli])
            return carry
        lax.fori_loop(0, CT, addbody, 0)

        def bbody(bl):
            blv = jnp.full((16,), 0, jnp.int32) + bl

            def sbody(sl):
                i = sl * BW + bl
                slv = jnp.full((16,), 0, jnp.int32) + sl
                for k in range(4):
                    plsc.store_scatter(
                        vb, [slv, dt_vecs[k], di_vec, blv],
                        gb[i, pl.ds(k * 16, 16)])
            plsc.parallel_loop(0, SC_, 1, unroll=4)(sbody)
        plsc.parallel_loop(0, BW, 1, unroll=2)(bbody)

    gd = [None, None]
    sd = [None, None]
    for c in range(NCHUNK + 1):
        if c < NCHUNK:
            b = c % 2
            if sd[b] is not None:
                sd[b].wait()
            prep_clist(c)
            gd[b] = start_gather(c)
        if c >= 1:
            cp = c - 1
            bp = cp % 2
            for d in gd[bp]:
                d.wait()
            compute(cp)
            sd[bp] = pltpu.async_copy(
                vbs[bp],
                out_hbm.at[pl.ds(cp * SC_, SC_), :, bt0, :, pl.ds(bi0, BW)],
                ssems[bp])
    sd[(NCHUNK - 2) % 2].wait()
    sd[(NCHUNK - 1) % 2].wait()


def kernel(x, token_table, pos_table):
    xf = x.reshape(B * S).astype(jnp.int32)
    out5 = _embed(xf, token_table, pos_table)
    out = out5.transpose(0, 1, 3, 2, 4).reshape(S, D, B).transpose(2, 0, 1)
    return out


# speedup vs baseline: 1.2209x; 1.2209x over previous
import functools

import jax
import jax.numpy as jnp
from jax import lax
from jax.experimental import pallas as pl
from jax.experimental.pallas import tpu as pltpu
from jax.experimental.pallas import tpu_sc as plsc

B, S, D = 1024, 200, 64

_info = plsc.get_sparse_core_info()
NC, NS = _info.num_cores, _info.num_subcores
NW = NC * NS
SEQ_W = B // NW
ROWS_W = SEQ_W * S

_SPLIT = 128

_mesh = plsc.VectorSubcoreMesh(core_axis_name="c", subcore_axis_name="s")


@functools.partial(
    pl.kernel,
    out_type=jax.ShapeDtypeStruct((B * S, D), jnp.float32),
    mesh=_mesh,
    compiler_params=pltpu.CompilerParams(use_tc_tiling_on_sc=False),
    scratch_types=[
        pltpu.VMEM((ROWS_W,), jnp.int32),
        pltpu.VMEM((S, D), jnp.float32),
        pltpu.VMEM((S, D), jnp.float32),
        pltpu.VMEM((S, D), jnp.float32),
        pltpu.SemaphoreType.DMA,
        pltpu.SemaphoreType.DMA,
        pltpu.SemaphoreType.DMA,
        pltpu.SemaphoreType.DMA,
    ],
)
def _embed(x_hbm, tok_hbm, pos_hbm, out_hbm, idx_v, pos_v, buf0, buf1,
           gsem0, gsem1, ssem0, ssem1):
    wid = lax.axis_index("s") * NC + lax.axis_index("c")
    base = wid * ROWS_W

    pltpu.sync_copy(pos_hbm, pos_v)
    pltpu.sync_copy(x_hbm.at[pl.ds(base, ROWS_W)], idx_v)

    bufs = (buf0, buf1)
    gsems = (gsem0, gsem1)
    ssems = (ssem0, ssem1)

    def start_gather(s):
        b = s % 2
        d0 = pltpu.async_copy(
            tok_hbm.at[idx_v.at[pl.ds(s * S, _SPLIT)]],
            bufs[b].at[pl.ds(0, _SPLIT)], gsems[b])
        d1 = pltpu.async_copy(
            tok_hbm.at[idx_v.at[pl.ds(s * S + _SPLIT, S - _SPLIT)]],
            bufs[b].at[pl.ds(_SPLIT, S - _SPLIT)], gsems[b])
        return (d0, d1)

    def add_pos(buf):
        def body(r, carry):
            for k in range(D // 16):
                sl = pl.ds(k * 16, 16)
                plsc.addupdate(buf.at[r, sl], pos_v[r, sl])
            return carry
        lax.fori_loop(0, S, body, 0)

    gd = [None, None]
    sd = [None, None]
    for s in range(SEQ_W + 1):
        if s < SEQ_W:
            b = s % 2
            if sd[b] is not None:
                sd[b].wait()
            gd[b] = start_gather(s)
        if s >= 1:
            sp = s - 1
            bp = sp % 2
            for d in gd[bp]:
                d.wait()
            add_pos(bufs[bp])
            sd[bp] = pltpu.async_copy(
                bufs[bp], out_hbm.at[pl.ds(base + sp * S, S)], ssems[bp])
    sd[(SEQ_W - 2) % 2].wait()
    sd[(SEQ_W - 1) % 2].wait()


def kernel(x, token_table, pos_table):
    xf = x.reshape(B * S).astype(jnp.int32)
    out = _embed(xf, token_table, pos_table)
    return out.reshape(B, S, D)
